# Initial kernel scaffold; baseline (speedup 1.0000x reference)
import jax
import jax.numpy as jnp
from jax.experimental import pallas as pl


def kernel(srcs, dsts, negs, timestamps, edge_feat, memory, time_w, time_b, W_ih, W_hh, b_ih, b_hh, W_emb, src_W, src_b, dst_W, dst_b, out_W, out_b):
    raise NotImplementedError("write your pallas kernel here")


# speedup vs baseline: 1.5998x; 1.5998x over previous
import functools

import jax
import jax.numpy as jnp
from jax import lax
from jax.experimental import pallas as pl
from jax.experimental.pallas import tpu as pltpu
from jax.experimental.pallas import tpu_sc as plsc

N = 100000
DM = 128
DE = 172
DT = 100
DEMB = 128
B = 16384
NIDS = 2 * B

NC = 2
NS = 16
NW = NC * NS
L = 16

_SC_MESH = dict(core_axis_name="c", subcore_axis_name="s")


def _wid():
  return lax.axis_index("c") * NS + lax.axis_index("s")


_W_CHUNK = 4096


@functools.partial(
    pl.kernel,
    out_type=jax.ShapeDtypeStruct((N,), jnp.int32),
    mesh=plsc.VectorSubcoreMesh(**_SC_MESH),
    compiler_params=pltpu.CompilerParams(needs_layout_passes=False),
    scratch_types=[
        pltpu.VMEM((_W_CHUNK,), jnp.int32),
        pltpu.VMEM((N,), jnp.int32),
    ],
)
def _winner_kernel(ids_hbm, p_hbm, ids_v, p_v):
  @pl.when(_wid() == 0)
  def _():
    def init_body(i, _):
      p_v[pl.ds(i * L, L)] = jnp.full((L,), -1, jnp.int32)
      return 0

    lax.fori_loop(0, N // L, init_body, 0)

    lane = lax.iota(jnp.int32, L)

    def chunk_body(c, _):
      pltpu.sync_copy(ids_hbm.at[pl.ds(c * _W_CHUNK, _W_CHUNK)], ids_v)

      def grp_body(g, _):
        v = ids_v[pl.ds(g * L, L)]
        pos = c * _W_CHUNK + g * L + lane
        _, is_last = plsc.scan_count(v)
        plsc.store_scatter(p_v, [v], pos, mask=is_last)
        return 0

      lax.fori_loop(0, _W_CHUNK // L, grp_body, 0)
      return 0

    lax.fori_loop(0, NIDS // _W_CHUNK, chunk_body, 0)
    pltpu.sync_copy(p_v, p_hbm)


_G_CHUNK = 128
_G_PER_W = NIDS // NW


@functools.partial(
    pl.kernel,
    out_type=jax.ShapeDtypeStruct((NIDS, DM), jnp.float32),
    mesh=plsc.VectorSubcoreMesh(**_SC_MESH),
    compiler_params=pltpu.CompilerParams(needs_layout_passes=False),
    scratch_types=[
        pltpu.VMEM((_G_CHUNK,), jnp.int32),
        pltpu.VMEM((_G_CHUNK, DM), jnp.float32),
        pltpu.SemaphoreType.DMA,
    ],
)
def _gather_kernel(mem_hbm, ids_hbm, out_hbm, idx_v, rows_v, sem):
  base_w = _wid() * _G_PER_W

  def chunk_body(c, _):
    base = base_w + c * _G_CHUNK
    pltpu.sync_copy(ids_hbm.at[pl.ds(base, _G_CHUNK)], idx_v)
    pltpu.async_copy(mem_hbm.at[idx_v], rows_v, sem).wait()
    pltpu.sync_copy(rows_v, out_hbm.at[pl.ds(base, _G_CHUNK)])
    return 0

  lax.fori_loop(0, _G_PER_W // _G_CHUNK, chunk_body, 0)


_R_CHUNK = 128
_R_FULL = N // _R_CHUNK
_R_TAIL = N - _R_FULL * _R_CHUNK
_R_ITER = (_R_FULL + NW - 1) // NW
_N_PER_W = B // NW
_N_CHUNK = 128


@functools.partial(
    pl.kernel,
    out_type=(
        jax.ShapeDtypeStruct((N, DM), jnp.float32),
        jax.ShapeDtypeStruct((B, DM), jnp.float32),
    ),
    mesh=plsc.VectorSubcoreMesh(**_SC_MESH),
    compiler_params=pltpu.CompilerParams(needs_layout_passes=False),
    scratch_types=[
        pltpu.VMEM((_R_CHUNK,), jnp.int32),
        pltpu.VMEM((_R_CHUNK,), jnp.int32),
        pltpu.VMEM((_R_CHUNK, DM), jnp.float32),
        pltpu.VMEM((_R_CHUNK, DM), jnp.float32),
        pltpu.SemaphoreType.DMA,
        pltpu.SemaphoreType.DMA,
    ],
)
def _resolve_kernel(mem_hbm, h_hbm, p_hbm, negs_hbm, newmem_hbm, mn_hbm,
                    p_c, idx_c, a_rows, h_rows, sem_a, sem_h):
  wid = _wid()
  lane = lax.iota(jnp.int32, L)

  def overlay(nrows, spread):
    def idx_body(g, _):
      p = p_c[pl.ds(g * L, L)]
      fallback = (spread + g * L + lane) & (NIDS - 1)
      idx_c[pl.ds(g * L, L)] = jnp.where(p >= 0, p, fallback)
      return 0

    lax.fori_loop(0, nrows // L, idx_body, 0)
    pltpu.async_copy(h_hbm.at[idx_c.at[pl.ds(0, nrows)]],
                     h_rows.at[pl.ds(0, nrows)], sem_h).wait()

    def sel_body(g, _):
      pv = p_c[pl.ds(g * L, L)]
      for l in range(L):
        i = g * L + l

        @pl.when(pv[l] >= 0)
        def _():
          for j in range(DM // L):
            a_rows[i, pl.ds(j * L, L)] = h_rows[i, pl.ds(j * L, L)]

      return 0

    lax.fori_loop(0, nrows // L, sel_body, 0)

  def chunk_body(k, _):
    g = wid + k * NW

    @pl.when(g < _R_FULL)
    def _():
      base = g * _R_CHUNK
      pltpu.sync_copy(p_hbm.at[pl.ds(base, _R_CHUNK)], p_c)
      pltpu.sync_copy(mem_hbm.at[pl.ds(base, _R_CHUNK)], a_rows)
      overlay(_R_CHUNK, base)
      pltpu.sync_copy(a_rows, newmem_hbm.at[pl.ds(base, _R_CHUNK)])

    return 0

  lax.fori_loop(0, _R_ITER, chunk_body, 0)

  @pl.when(wid == 0)
  def _():
    base = _R_FULL * _R_CHUNK
    pltpu.sync_copy(p_hbm.at[pl.ds(base, _R_TAIL)],
                    p_c.at[pl.ds(0, _R_TAIL)])
    pltpu.sync_copy(mem_hbm.at[pl.ds(base, _R_TAIL)],
                    a_rows.at[pl.ds(0, _R_TAIL)])
    overlay(_R_TAIL, base)
    pltpu.sync_copy(a_rows.at[pl.ds(0, _R_TAIL)],
                    newmem_hbm.at[pl.ds(base, _R_TAIL)])

  def neg_body(c, _):
    base = wid * _N_PER_W + c * _N_CHUNK
    pltpu.sync_copy(negs_hbm.at[pl.ds(base, _N_CHUNK)], idx_c)
    pltpu.async_copy(p_hbm.at[idx_c], p_c, sem_a).wait()
    pltpu.async_copy(mem_hbm.at[idx_c], a_rows, sem_a).wait()
    overlay(_N_CHUNK, base)
    pltpu.sync_copy(a_rows, mn_hbm.at[pl.ds(base, _N_CHUNK)])
    return 0

  lax.fori_loop(0, _N_PER_W // _N_CHUNK, neg_body, 0)


_BM = 1024


def _dotT(x, w):
  return lax.dot_general(x, w, (((1,), (1,)), ((), ())),
                         preferred_element_type=jnp.float32)


def _gru_block(msd_ref, ef_ref, ts_ref, tw_ref, tb_ref, wih_ref, whh_ref,
               bih_ref, bhh_ref, wemb_ref, srcw_ref, srcb_ref, dstw_ref,
               dstb_ref, outw_ref, outb_ref, h_ref, as_ref, pp_ref):
  m_s = msd_ref[0]
  m_d = msd_ref[1]
  te = jnp.cos(ts_ref[...] * tw_ref[...] + tb_ref[...])

  wih = wih_ref[...]
  w1 = wih[:, :DM]
  w2 = wih[:, DM:2 * DM]
  w3 = wih[:, 2 * DM:2 * DM + DE]
  w4 = wih[:, 2 * DM + DE:]
  shared = _dotT(ef_ref[...], w3) + _dotT(te, w4) + bih_ref[...]

  a1_s = _dotT(m_s, w1)
  a2_s = _dotT(m_d, w2)
  a1_d = _dotT(m_d, w1)
  a2_d = _dotT(m_s, w2)

  whh = whh_ref[...]
  bhh = bhh_ref[...]
  gh_s = _dotT(m_s, whh) + bhh
  gh_d = _dotT(m_d, whh) + bhh

  def gru(gi, gh, hprev):
    r = jax.nn.sigmoid(gi[:, :DM] + gh[:, :DM])
    z = jax.nn.sigmoid(gi[:, DM:2 * DM] + gh[:, DM:2 * DM])
    n = jnp.tanh(gi[:, 2 * DM:] + r * gh[:, 2 * DM:])
    return (1.0 - z) * n + z * hprev

  h_s = gru(shared + a1_s + a2_s, gh_s, m_s)
  h_d = gru(shared + a1_d + a2_d, gh_d, m_d)
  h_ref[0] = h_s
  h_ref[1] = h_d

  z_s = _dotT(h_s, wemb_ref[...])
  z_d = _dotT(h_d, wemb_ref[...])
  a_s = _dotT(z_s, srcw_ref[...]) + srcb_ref[...]
  as_ref[...] = a_s
  bz_d = _dotT(z_d, dstw_ref[...]) + dstb_ref[...]
  hp = jax.nn.relu(a_s + bz_d)
  pp_ref[...] = jnp.sum(hp * outw_ref[...], axis=1, keepdims=True) + outb_ref[0, 0]


def _gru_call(m_sd3, edge_feat, ts2, tw2, tb2, W_ih, W_hh, bih2, bhh2,
              W_emb, src_W, srcb2, dst_W, dstb2, out_W, outb2):
  grid = (B // _BM,)
  full = lambda shape: pl.BlockSpec(shape, lambda g: (0,) * len(shape))
  return pl.pallas_call(
      _gru_block,
      grid=grid,
      in_specs=[
          pl.BlockSpec((2, _BM, DM), lambda g: (0, g, 0)),
          pl.BlockSpec((_BM, DE), lambda g: (g, 0)),
          pl.BlockSpec((_BM, 1), lambda g: (g, 0)),
          full((1, DT)),
          full((1, DT)),
          full((3 * DM, 2 * DM + DE + DT)),
          full((3 * DM, DM)),
          full((1, 3 * DM)),
          full((1, 3 * DM)),
          full((DEMB, DM)),
          full((DEMB, DEMB)),
          full((1, DEMB)),
          full((DEMB, DEMB)),
          full((1, DEMB)),
          full((1, DEMB)),
          full((1, 1)),
      ],
      out_specs=[
          pl.BlockSpec((2, _BM, DM), lambda g: (0, g, 0)),
          pl.BlockSpec((_BM, DEMB), lambda g: (g, 0)),
          pl.BlockSpec((_BM, 1), lambda g: (g, 0)),
      ],
      out_shape=[
          jax.ShapeDtypeStruct((2, B, DM), jnp.float32),
          jax.ShapeDtypeStruct((B, DEMB), jnp.float32),
          jax.ShapeDtypeStruct((B, 1), jnp.float32),
      ],
  )(m_sd3, edge_feat, ts2, tw2, tb2, W_ih, W_hh, bih2, bhh2, W_emb, src_W,
    srcb2, dst_W, dstb2, out_W, outb2)


def _neg_block(mn_ref, as_ref, wemb_ref, dstw_ref, dstb_ref, outw_ref,
               outb_ref, pn_ref):
  z_n = _dotT(mn_ref[...], wemb_ref[...])
  bz_n = _dotT(z_n, dstw_ref[...]) + dstb_ref[...]
  hp = jax.nn.relu(as_ref[...] + bz_n)
  pn_ref[...] = jnp.sum(hp * outw_ref[...], axis=1, keepdims=True) + outb_ref[0, 0]


def _neg_call(m_n, a_s, W_emb, dst_W, dstb2, out_W, outb2):
  grid = (B // _BM,)
  full = lambda shape: pl.BlockSpec(shape, lambda g: (0,) * len(shape))
  return pl.pallas_call(
      _neg_block,
      grid=grid,
      in_specs=[
          pl.BlockSpec((_BM, DM), lambda g: (g, 0)),
          pl.BlockSpec((_BM, DEMB), lambda g: (g, 0)),
          full((DEMB, DM)),
          full((DEMB, DEMB)),
          full((1, DEMB)),
          full((1, DEMB)),
          full((1, 1)),
      ],
      out_specs=pl.BlockSpec((_BM, 1), lambda g: (g, 0)),
      out_shape=jax.ShapeDtypeStruct((B, 1), jnp.float32),
  )(m_n, a_s, W_emb, dst_W, dstb2, out_W, outb2)


def kernel(srcs, dsts, negs, timestamps, edge_feat, memory, time_w, time_b,
           W_ih, W_hh, b_ih, b_hh, W_emb, src_W, src_b, dst_W, dst_b,
           out_W, out_b):
  srcs = srcs.astype(jnp.int32)
  dsts = dsts.astype(jnp.int32)
  negs = negs.astype(jnp.int32)
  ids = jnp.concatenate([srcs, dsts])

  p_tbl = _winner_kernel(ids)
  m_sd = _gather_kernel(memory, ids)
  m_sd3 = m_sd.reshape(2, B, DM)

  ts2 = timestamps.reshape(B, 1)
  tw2 = time_w.reshape(1, DT)
  tb2 = time_b.reshape(1, DT)
  bih2 = b_ih.reshape(1, 3 * DM)
  bhh2 = b_hh.reshape(1, 3 * DM)
  srcb2 = src_b.reshape(1, DEMB)
  dstb2 = dst_b.reshape(1, DEMB)
  outb2 = out_b.reshape(1, 1)

  h3, a_s, pp = _gru_call(m_sd3, edge_feat, ts2, tw2, tb2, W_ih, W_hh, bih2,
                          bhh2, W_emb, src_W, srcb2, dst_W, dstb2, out_W,
                          outb2)
  h = h3.reshape(NIDS, DM)

  new_memory, m_n = _resolve_kernel(memory, h, p_tbl, negs)
  pn = _neg_call(m_n, a_s, W_emb, dst_W, dstb2, out_W, outb2)

  return pp[:, 0], pn[:, 0], new_memory
